# Initial kernel scaffold; baseline (speedup 1.0000x reference)
import jax
import jax.numpy as jnp
from jax.experimental import pallas as pl


def kernel(x, edge_index, W1l, b1l, W1r, W2l, b2l, W2r):
    raise NotImplementedError("write your pallas kernel here")


# speedup vs baseline: 9.7979x; 9.7979x over previous
import functools

import jax
import jax.numpy as jnp
from jax import lax
from jax.experimental import pallas as pl
from jax.experimental.pallas import tpu as pltpu
from jax.experimental.pallas import tpu_sc as plsc

N = 10000
E = 320000
F = 128
PW = 16
NC, NS = 2, 16
NW = NC * NS
EPT = E // NW
CH = 100
NCH = EPT // CH
ZCH = 100
NZ = N // ZCH
ZPT = -(-NZ // NS)


def _mesh():
    return plsc.VectorSubcoreMesh(
        core_axis_name="c", subcore_axis_name="s", num_cores=NC, num_subcores=NS
    )


@functools.cache
def _sc_agg_wide():
    return pl.kernel(
        _sc_agg_wide_body,
        out_type=(
            jax.ShapeDtypeStruct((NC, N, F), jnp.float32),
            jax.ShapeDtypeStruct((NC, N, PW), jnp.float32),
        ),
        mesh=_mesh(),
        scratch_types=[
            pltpu.VMEM((NCH, CH), jnp.int32),
            pltpu.VMEM((NCH, CH), jnp.int32),
            pltpu.VMEM((CH, PW), jnp.float32),
            pltpu.VMEM((CH, F), jnp.float32),
            pltpu.VMEM((ZCH, PW), jnp.float32),
            pltpu.VMEM_SHARED((N, F), jnp.float32),
            pltpu.VMEM_SHARED((N, PW), jnp.float32),
            pltpu.SemaphoreType.DMA,
        ],
        compiler_params=pltpu.CompilerParams(use_tc_tiling_on_sc=False),
    )


def _sc_agg_wide_body(x_hbm, src_hbm, dst_hbm, ones_hbm, z128_hbm, z16_hbm,
                      acc_out, cnt_out,
                      src_v, dst_v, ones_v, rows_v, cb_v, acc_sh, cnt_sh,
                      sem):
    cid = lax.axis_index("c")
    sid = lax.axis_index("s")
    wid = sid * NC + cid

    pltpu.sync_copy(z128_hbm, rows_v)
    pltpu.sync_copy(z16_hbm, cb_v)
    for j in range(ZPT):
        k = sid + NS * j

        @pl.when(k < NZ)
        def _():
            pltpu.sync_copy(rows_v, acc_sh.at[pl.ds(k * ZCH, ZCH)])
            pltpu.sync_copy(cb_v, cnt_sh.at[pl.ds(k * ZCH, ZCH)])

    pltpu.sync_copy(src_hbm.at[wid], src_v)
    pltpu.sync_copy(dst_hbm.at[wid], dst_v)
    pltpu.sync_copy(ones_hbm, ones_v)
    plsc.subcore_barrier()

    def chunk(ch, carry):
        pltpu.async_copy(x_hbm.at[src_v.at[ch]], rows_v, sem).wait()
        pltpu.sync_copy(rows_v, acc_sh.at[dst_v.at[ch]], add=True)
        pltpu.sync_copy(ones_v, cnt_sh.at[dst_v.at[ch]], add=True)
        return carry

    lax.fori_loop(0, NCH, chunk, 0)
    plsc.subcore_barrier()

    for j in range(ZPT):
        k = sid + NS * j

        @pl.when(k < NZ)
        def _():
            pltpu.sync_copy(acc_sh.at[pl.ds(k * ZCH, ZCH)], rows_v)
            pltpu.sync_copy(rows_v, acc_out.at[cid, pl.ds(k * ZCH, ZCH)])
            pltpu.sync_copy(cnt_sh.at[pl.ds(k * ZCH, ZCH)], cb_v)
            pltpu.sync_copy(cb_v, cnt_out.at[cid, pl.ds(k * ZCH, ZCH)])


@functools.cache
def _sc_agg_narrow():
    return pl.kernel(
        _sc_agg_narrow_body,
        out_type=jax.ShapeDtypeStruct((NC, N, PW), jnp.float32),
        mesh=_mesh(),
        scratch_types=[
            pltpu.VMEM((NCH, CH), jnp.int32),
            pltpu.VMEM((NCH, CH), jnp.int32),
            pltpu.VMEM((CH, PW), jnp.float32),
            pltpu.VMEM((ZCH, PW), jnp.float32),
            pltpu.VMEM_SHARED((N, PW), jnp.float32),
            pltpu.SemaphoreType.DMA,
        ],
        compiler_params=pltpu.CompilerParams(use_tc_tiling_on_sc=False),
    )


def _sc_agg_narrow_body(p_hbm, src_hbm, dst_hbm, z16_hbm, acc_out,
                        src_v, dst_v, rows_v, cb_v, acc_sh, sem):
    cid = lax.axis_index("c")
    sid = lax.axis_index("s")
    wid = sid * NC + cid

    pltpu.sync_copy(z16_hbm, cb_v)
    for j in range(ZPT):
        k = sid + NS * j

        @pl.when(k < NZ)
        def _():
            pltpu.sync_copy(cb_v, acc_sh.at[pl.ds(k * ZCH, ZCH)])

    pltpu.sync_copy(src_hbm.at[wid], src_v)
    pltpu.sync_copy(dst_hbm.at[wid], dst_v)
    plsc.subcore_barrier()

    def chunk(ch, carry):
        pltpu.async_copy(p_hbm.at[src_v.at[ch]], rows_v, sem).wait()
        pltpu.sync_copy(rows_v, acc_sh.at[dst_v.at[ch]], add=True)
        return carry

    lax.fori_loop(0, NCH, chunk, 0)
    plsc.subcore_barrier()

    for j in range(ZPT):
        k = sid + NS * j

        @pl.when(k < NZ)
        def _():
            pltpu.sync_copy(acc_sh.at[pl.ds(k * ZCH, ZCH)], rows_v)
            pltpu.sync_copy(rows_v, acc_out.at[cid, pl.ds(k * ZCH, ZCH)])


RBLK = 1000


def _tc1_body(accp, cntp, x, w1lt, b1l, w1rt, w2t, h_out, p_out):
    a = accp[...]
    c = cntp[...]
    acc = a[0] + a[1]
    cnt = jnp.maximum(c[0, :, 0:1] + c[1, :, 0:1], 1.0)
    agg = acc / cnt
    h = (
        jnp.dot(agg, w1lt[...], preferred_element_type=jnp.float32)
        + b1l[...]
        + jnp.dot(x[...], w1rt[...], preferred_element_type=jnp.float32)
    )
    h = jnp.maximum(h, 0.0)
    h_out[...] = h
    p_out[...] = jnp.dot(h, w2t[...], preferred_element_type=jnp.float32)


def _tc1(accp, cntp, x, w1lt, b1l, w1rt, w2t, interpret=False):
    return pl.pallas_call(
        _tc1_body,
        grid=(N // RBLK,),
        in_specs=[
            pl.BlockSpec((NC, RBLK, F), lambda i: (0, i, 0)),
            pl.BlockSpec((NC, RBLK, PW), lambda i: (0, i, 0)),
            pl.BlockSpec((RBLK, F), lambda i: (i, 0)),
            pl.BlockSpec((F, F), lambda i: (0, 0)),
            pl.BlockSpec((1, F), lambda i: (0, 0)),
            pl.BlockSpec((F, F), lambda i: (0, 0)),
            pl.BlockSpec((F, PW), lambda i: (0, 0)),
        ],
        out_specs=[
            pl.BlockSpec((RBLK, F), lambda i: (i, 0)),
            pl.BlockSpec((RBLK, PW), lambda i: (i, 0)),
        ],
        out_shape=[
            jax.ShapeDtypeStruct((N, F), jnp.float32),
            jax.ShapeDtypeStruct((N, PW), jnp.float32),
        ],
        interpret=interpret,
    )(accp, cntp, x, w1lt, b1l, w1rt, w2t)


def _tc2_body(acc2p, cntp, p, b2l, out_ref):
    a = acc2p[...]
    c = cntp[...]
    s = a[0, :, 0:1] + a[1, :, 0:1]
    cnt = jnp.maximum(c[0, :, 0:1] + c[1, :, 0:1], 1.0)
    z = s / cnt + b2l[0, 0] + p[:, 1:2]
    out_ref[...] = jax.nn.sigmoid(z)


def _tc2(acc2p, cntp, p, b2l, interpret=False):
    return pl.pallas_call(
        _tc2_body,
        grid=(N // RBLK,),
        in_specs=[
            pl.BlockSpec((NC, RBLK, PW), lambda i: (0, i, 0)),
            pl.BlockSpec((NC, RBLK, PW), lambda i: (0, i, 0)),
            pl.BlockSpec((RBLK, PW), lambda i: (i, 0)),
            pl.BlockSpec((1, 1), lambda i: (0, 0)),
        ],
        out_specs=pl.BlockSpec((RBLK, 1), lambda i: (i, 0)),
        out_shape=jax.ShapeDtypeStruct((N, 1), jnp.float32),
        interpret=interpret,
    )(acc2p, cntp, p, b2l)


def kernel(x, edge_index, W1l, b1l, W1r, W2l, b2l, W2r):
    ei = edge_index.astype(jnp.int32)
    src = ei[0].reshape(NW, NCH, CH)
    dst = ei[1].reshape(NW, NCH, CH)
    ones = jnp.ones((CH, PW), jnp.float32)
    z128 = jnp.zeros((ZCH, F), jnp.float32)
    z16 = jnp.zeros((ZCH, PW), jnp.float32)

    accp, cntp = _sc_agg_wide()(x, src, dst, ones, z128, z16)

    w1lt = W1l.T
    w1rt = W1r.T
    b1l2 = b1l.reshape(1, F)
    w2 = jnp.zeros((PW, F), jnp.float32).at[0].set(W2l[0]).at[1].set(W2r[0])
    w2t = w2.T

    h, p = _tc1(accp, cntp, x, w1lt, b1l2, w1rt, w2t)

    acc2p = _sc_agg_narrow()(p, src, dst, z16)

    out = _tc2(acc2p, cntp, p, b2l.reshape(1, 1))
    return (out, h)


# speedup vs baseline: 11.8714x; 1.2116x over previous
import functools

import jax
import jax.numpy as jnp
from jax import lax
from jax.experimental import pallas as pl
from jax.experimental.pallas import tpu as pltpu
from jax.experimental.pallas import tpu_sc as plsc

N = 10000
E = 320000
F = 128
PW = 16
NC, NS = 2, 16
NW = NC * NS
EPT = E // NW
CH = 100
NCH = EPT // CH
ZCH = 100
NZ = N // ZCH
ZPT = -(-NZ // NS)


def _mesh():
    return plsc.VectorSubcoreMesh(
        core_axis_name="c", subcore_axis_name="s", num_cores=NC, num_subcores=NS
    )


def _pipelined_agg(x_hbm, eidx_hbm, wid, rows, ebs, acc_sh, gsem, ssem, isem,
                   extra_scatter=None):
    for b in (0, 1):
        pltpu.sync_copy(eidx_hbm.at[wid, b], ebs[b])
        pltpu.async_copy(x_hbm.at[ebs[b].at[0]], rows[b], gsem[b])

    def step(g, carry):
        for b in (0, 1):
            c = 2 * g + b
            pltpu.make_async_copy(x_hbm.at[ebs[b].at[0]], rows[b],
                                  gsem[b]).wait()
            sdesc = pltpu.async_copy(rows[b], acc_sh.at[ebs[b].at[1]],
                                     ssem[b], add=True)
            odesc = None
            if extra_scatter is not None:
                odesc = extra_scatter(b, ebs[b].at[1])
            sdesc.wait()
            if odesc is not None:
                odesc.wait()

            @pl.when(g < NCH // 2 - 1)
            def _():
                pltpu.async_copy(eidx_hbm.at[wid, c + 2], ebs[b],
                                 isem[b]).wait()
                pltpu.async_copy(x_hbm.at[ebs[b].at[0]], rows[b], gsem[b])

        return carry

    lax.fori_loop(0, NCH // 2, step, 0)


@functools.cache
def _sc_agg_wide():
    return pl.kernel(
        _sc_agg_wide_body,
        out_type=(
            jax.ShapeDtypeStruct((NC, N, F), jnp.float32),
            jax.ShapeDtypeStruct((NC, N, PW), jnp.float32),
        ),
        mesh=_mesh(),
        scratch_types=[
            pltpu.VMEM((2, CH), jnp.int32),
            pltpu.VMEM((2, CH), jnp.int32),
            pltpu.VMEM((CH, F), jnp.float32),
            pltpu.VMEM((CH, F), jnp.float32),
            pltpu.VMEM((CH, PW), jnp.float32),
            pltpu.VMEM((ZCH, PW), jnp.float32),
            pltpu.VMEM_SHARED((N, F), jnp.float32),
            pltpu.VMEM_SHARED((N, PW), jnp.float32),
            pltpu.SemaphoreType.DMA,
            pltpu.SemaphoreType.DMA,
            pltpu.SemaphoreType.DMA,
            pltpu.SemaphoreType.DMA,
            pltpu.SemaphoreType.DMA,
            pltpu.SemaphoreType.DMA,
            pltpu.SemaphoreType.DMA,
            pltpu.SemaphoreType.DMA,
        ],
        compiler_params=pltpu.CompilerParams(use_tc_tiling_on_sc=False),
    )


def _sc_agg_wide_body(x_hbm, eidx_hbm, ones_hbm, z128_hbm, z16_hbm,
                      acc_out, cnt_out,
                      eb0, eb1, rows0, rows1, ones_v, cb_v, acc_sh, cnt_sh,
                      gsem0, gsem1, ssem0, ssem1, isem0, isem1, osem0, osem1):
    cid = lax.axis_index("c")
    sid = lax.axis_index("s")
    wid = sid * NC + cid

    pltpu.sync_copy(z128_hbm, rows0)
    pltpu.sync_copy(z16_hbm, cb_v)
    pltpu.sync_copy(ones_hbm, ones_v)
    for j in range(ZPT):
        k = sid + NS * j

        @pl.when(k < NZ)
        def _():
            pltpu.sync_copy(rows0, acc_sh.at[pl.ds(k * ZCH, ZCH)])
            pltpu.sync_copy(cb_v, cnt_sh.at[pl.ds(k * ZCH, ZCH)])

    plsc.subcore_barrier()

    osems = (osem0, osem1)

    def ones_scatter(b, dst_idx):
        return pltpu.async_copy(ones_v, cnt_sh.at[dst_idx], osems[b],
                                add=True)

    _pipelined_agg(x_hbm, eidx_hbm, wid, (rows0, rows1), (eb0, eb1), acc_sh,
                   (gsem0, gsem1), (ssem0, ssem1), (isem0, isem1),
                   extra_scatter=ones_scatter)
    plsc.subcore_barrier()

    for j in range(ZPT):
        k = sid + NS * j

        @pl.when(k < NZ)
        def _():
            pltpu.sync_copy(acc_sh.at[pl.ds(k * ZCH, ZCH)], rows0)
            pltpu.sync_copy(rows0, acc_out.at[cid, pl.ds(k * ZCH, ZCH)])
            pltpu.sync_copy(cnt_sh.at[pl.ds(k * ZCH, ZCH)], cb_v)
            pltpu.sync_copy(cb_v, cnt_out.at[cid, pl.ds(k * ZCH, ZCH)])


@functools.cache
def _sc_agg_narrow():
    return pl.kernel(
        _sc_agg_narrow_body,
        out_type=jax.ShapeDtypeStruct((NC, N, PW), jnp.float32),
        mesh=_mesh(),
        scratch_types=[
            pltpu.VMEM((2, CH), jnp.int32),
            pltpu.VMEM((2, CH), jnp.int32),
            pltpu.VMEM((CH, PW), jnp.float32),
            pltpu.VMEM((CH, PW), jnp.float32),
            pltpu.VMEM((ZCH, PW), jnp.float32),
            pltpu.VMEM_SHARED((N, PW), jnp.float32),
            pltpu.SemaphoreType.DMA,
            pltpu.SemaphoreType.DMA,
            pltpu.SemaphoreType.DMA,
            pltpu.SemaphoreType.DMA,
            pltpu.SemaphoreType.DMA,
            pltpu.SemaphoreType.DMA,
        ],
        compiler_params=pltpu.CompilerParams(use_tc_tiling_on_sc=False),
    )


def _sc_agg_narrow_body(p_hbm, eidx_hbm, z16_hbm, acc_out,
                        eb0, eb1, rows0, rows1, cb_v, acc_sh,
                        gsem0, gsem1, ssem0, ssem1, isem0, isem1):
    cid = lax.axis_index("c")
    sid = lax.axis_index("s")
    wid = sid * NC + cid

    pltpu.sync_copy(z16_hbm, cb_v)
    for j in range(ZPT):
        k = sid + NS * j

        @pl.when(k < NZ)
        def _():
            pltpu.sync_copy(cb_v, acc_sh.at[pl.ds(k * ZCH, ZCH)])

    plsc.subcore_barrier()
    _pipelined_agg(p_hbm, eidx_hbm, wid, (rows0, rows1), (eb0, eb1), acc_sh,
                   (gsem0, gsem1), (ssem0, ssem1), (isem0, isem1))
    plsc.subcore_barrier()

    for j in range(ZPT):
        k = sid + NS * j

        @pl.when(k < NZ)
        def _():
            pltpu.sync_copy(acc_sh.at[pl.ds(k * ZCH, ZCH)], rows0)
            pltpu.sync_copy(rows0, acc_out.at[cid, pl.ds(k * ZCH, ZCH)])


RBLK = 1000


def _tc1_body(accp, cntp, x, w1l, b1l, w1r, w2, h_out, p_out):
    a = accp[...]
    c = cntp[...]
    acc = a[0] + a[1]
    cnt = jnp.maximum(c[0, :, 0:1] + c[1, :, 0:1], 1.0)
    agg = acc / cnt
    dn = (((1,), (1,)), ((), ()))
    h = (
        lax.dot_general(agg, w1l[...], dn, preferred_element_type=jnp.float32)
        + b1l[...]
        + lax.dot_general(x[...], w1r[...], dn,
                          preferred_element_type=jnp.float32)
    )
    h = jnp.maximum(h, 0.0)
    h_out[...] = h
    p_out[...] = lax.dot_general(h, w2[...], dn,
                                 preferred_element_type=jnp.float32)


def _tc1(accp, cntp, x, w1l, b1l, w1r, w2, interpret=False):
    return pl.pallas_call(
        _tc1_body,
        grid=(N // RBLK,),
        in_specs=[
            pl.BlockSpec((NC, RBLK, F), lambda i: (0, i, 0)),
            pl.BlockSpec((NC, RBLK, PW), lambda i: (0, i, 0)),
            pl.BlockSpec((RBLK, F), lambda i: (i, 0)),
            pl.BlockSpec((F, F), lambda i: (0, 0)),
            pl.BlockSpec((1, F), lambda i: (0, 0)),
            pl.BlockSpec((F, F), lambda i: (0, 0)),
            pl.BlockSpec((PW, F), lambda i: (0, 0)),
        ],
        out_specs=[
            pl.BlockSpec((RBLK, F), lambda i: (i, 0)),
            pl.BlockSpec((RBLK, PW), lambda i: (i, 0)),
        ],
        out_shape=[
            jax.ShapeDtypeStruct((N, F), jnp.float32),
            jax.ShapeDtypeStruct((N, PW), jnp.float32),
        ],
        interpret=interpret,
    )(accp, cntp, x, w1l, b1l, w1r, w2)


def _tc2_body(acc2p, cntp, p, b2l, out_ref):
    a = acc2p[...]
    c = cntp[...]
    s = a[0, :, 0:1] + a[1, :, 0:1]
    cnt = jnp.maximum(c[0, :, 0:1] + c[1, :, 0:1], 1.0)
    z = s / cnt + b2l[0, 0] + p[:, 1:2]
    out_ref[...] = jax.nn.sigmoid(z)


def _tc2(acc2p, cntp, p, b2l, interpret=False):
    return pl.pallas_call(
        _tc2_body,
        grid=(N // RBLK,),
        in_specs=[
            pl.BlockSpec((NC, RBLK, PW), lambda i: (0, i, 0)),
            pl.BlockSpec((NC, RBLK, PW), lambda i: (0, i, 0)),
            pl.BlockSpec((RBLK, PW), lambda i: (i, 0)),
            pl.BlockSpec((1, 1), lambda i: (0, 0)),
        ],
        out_specs=pl.BlockSpec((RBLK, 1), lambda i: (i, 0)),
        out_shape=jax.ShapeDtypeStruct((N, 1), jnp.float32),
        interpret=interpret,
    )(acc2p, cntp, p, b2l)


def kernel(x, edge_index, W1l, b1l, W1r, W2l, b2l, W2r):
    ei = edge_index.astype(jnp.int32)
    src = ei[0].reshape(NW, NCH, 1, CH)
    dst = ei[1].reshape(NW, NCH, 1, CH)
    eidx = jnp.concatenate([src, dst], axis=2)
    ones = jnp.ones((CH, PW), jnp.float32)
    z128 = jnp.zeros((ZCH, F), jnp.float32)
    z16 = jnp.zeros((ZCH, PW), jnp.float32)

    accp, cntp = _sc_agg_wide()(x, eidx, ones, z128, z16)

    b1l2 = b1l.reshape(1, F)
    w2 = jnp.concatenate([W2l, W2r, jnp.zeros((PW - 2, F), jnp.float32)], 0)

    h, p = _tc1(accp, cntp, x, W1l, b1l2, W1r, w2)

    acc2p = _sc_agg_narrow()(p, eidx, z16)

    out = _tc2(acc2p, cntp, p, b2l.reshape(1, 1))
    return (out, h)
